# baseline (speedup 1.0000x reference)
import jax
import jax.numpy as jnp
from jax.experimental import pallas as pl

_INTERPRET = False

_N = 4096
_BLK = 256
_M = 3


def _nn_body(s_ref, d_ref, dist_ref, g_ref):
    s = s_ref[...]
    dx = d_ref[0:1, :]
    dy = d_ref[1:2, :]
    dz = d_ref[2:3, :]
    tx = s[:, 0:1] - dx
    ty = s[:, 1:2] - dy
    tz = s[:, 2:3] - dz
    d2 = tx * tx + ty * ty + tz * tz
    mind = jnp.min(d2, axis=1, keepdims=True)
    iota = jax.lax.broadcasted_iota(jnp.int32, d2.shape, 1)
    bidx = jnp.min(jnp.where(d2 <= mind, iota, _N), axis=1, keepdims=True)
    onehot = iota == bidx
    gx = jnp.sum(jnp.where(onehot, dx, 0.0), axis=1)
    gy = jnp.sum(jnp.where(onehot, dy, 0.0), axis=1)
    gz = jnp.sum(jnp.where(onehot, dz, 0.0), axis=1)
    dist_ref[0, 0, :] = jnp.sqrt(jnp.maximum(mind[:, 0], 0.0))
    g_ref[0, 0, :] = gx
    g_ref[0, 1, :] = gy
    g_ref[0, 2, :] = gz


def _nn(src_pts, dstT):
    nblk = _N // _BLK
    dist, g = pl.pallas_call(
        _nn_body,
        grid=(nblk,),
        in_specs=[
            pl.BlockSpec((_BLK, _M), lambda i: (i, 0)),
            pl.BlockSpec((_M, _N), lambda i: (0, 0)),
        ],
        out_specs=[
            pl.BlockSpec((1, 1, _BLK), lambda i: (i, 0, 0)),
            pl.BlockSpec((1, _M, _BLK), lambda i: (i, 0, 0)),
        ],
        out_shape=[
            jax.ShapeDtypeStruct((nblk, 1, _BLK), jnp.float32),
            jax.ShapeDtypeStruct((nblk, _M, _BLK), jnp.float32),
        ],
        interpret=_INTERPRET,
    )(src_pts, dstT)
    dist = dist.reshape(-1)
    G = g.transpose(0, 2, 1).reshape(-1, _M)
    return dist, G


def _fit_from_moments(M, cA, cB, dtype):
    Sxx, Sxy, Sxz = M[0, 0], M[0, 1], M[0, 2]
    Syx, Syy, Syz = M[1, 0], M[1, 1], M[1, 2]
    Szx, Szy, Szz = M[2, 0], M[2, 1], M[2, 2]
    N4 = jnp.array([
        [Sxx + Syy + Szz, Syz - Szy, Szx - Sxz, Sxy - Syx],
        [Syz - Szy, Sxx - Syy - Szz, Sxy + Syx, Szx + Sxz],
        [Szx - Sxz, Sxy + Syx, -Sxx + Syy - Szz, Syz + Szy],
        [Sxy - Syx, Szx + Sxz, Syz + Szy, -Sxx - Syy + Szz]], dtype=dtype)
    V = jnp.eye(4, dtype=dtype)
    Nk = N4
    for _ in range(6):
        for (p, q) in ((0, 1), (0, 2), (0, 3), (1, 2), (1, 3), (2, 3)):
            apq = Nk[p, q]
            theta = 0.5 * jnp.arctan2(2.0 * apq, Nk[q, q] - Nk[p, p])
            z = jnp.abs(apq) > 0.0
            c = jnp.where(z, jnp.cos(theta), 1.0)
            s = jnp.where(z, jnp.sin(theta), 0.0)
            rp, rq = Nk[p, :], Nk[q, :]
            Nk = Nk.at[p, :].set(c * rp - s * rq).at[q, :].set(s * rp + c * rq)
            cp, cq = Nk[:, p], Nk[:, q]
            Nk = Nk.at[:, p].set(c * cp - s * cq).at[:, q].set(s * cp + c * cq)
            vp, vq = V[:, p], V[:, q]
            V = V.at[:, p].set(c * vp - s * vq).at[:, q].set(s * vp + c * vq)
    lam = jnp.diagonal(Nk)
    qv = V[:, jnp.argmax(lam)]
    w, x, y, z = qv[0], qv[1], qv[2], qv[3]
    R = jnp.array([
        [w * w + x * x - y * y - z * z, 2 * (x * y - w * z), 2 * (x * z + w * y)],
        [2 * (x * y + w * z), w * w - x * x + y * y - z * z, 2 * (y * z - w * x)],
        [2 * (x * z - w * y), 2 * (y * z + w * x), w * w - x * x - y * y + z * z]],
        dtype=dtype)
    t = cB - R @ cA
    return R, t


def _fit(A, B):
    m = A.shape[1]
    cA = jnp.mean(A, axis=0)
    cB = jnp.mean(B, axis=0)
    M = (A - cA).T @ (B - cB)
    R, t = _fit_from_moments(M, cA, cB, A.dtype)
    T = jnp.eye(m + 1, dtype=A.dtype)
    T = T.at[:m, :m].set(R)
    T = T.at[:m, m].set(t)
    return T, R, t


def kernel(A, B):
    max_iterations = 20
    tolerance = 0.001
    dstT = B.T

    def cond(c):
        _, _, done, i = c
        return jnp.logical_and(i < max_iterations, jnp.logical_not(done))

    def body(c):
        src, prev_error, done, i = c
        dist, G = _nn(src, dstT)
        _, R, t = _fit(src, G)
        src_new = src @ R.T + t
        mean_error = jnp.mean(dist)
        converged = jnp.abs(prev_error - mean_error) < tolerance
        return (src_new, mean_error, done | converged, i + 1)

    init = (A, jnp.zeros((), A.dtype), jnp.array(False), jnp.array(0, jnp.int32))
    src, _, _, _ = jax.lax.while_loop(cond, body, init)
    T, _, _ = _fit(A, src)
    return T


# speedup vs baseline: 5.4412x; 5.4412x over previous
import jax
import jax.numpy as jnp
from jax import lax
from jax.experimental import pallas as pl
from jax.experimental.pallas import tpu as pltpu
from jax.experimental.pallas import tpu_sc as plsc

_INTERPRET = False

_N = 4096
_BLK = 256
_NSUB = 16
_RS = _N // _NSUB
_F32 = jnp.float32


def _nn_body(sx_ref, sy_ref, sz_ref, d_ref, bidx_ref, dist_ref):
    sx = sx_ref[...][None, :]
    sy = sy_ref[...][None, :]
    sz = sz_ref[...][None, :]
    dx = d_ref[:, 0:1]
    dy = d_ref[:, 1:2]
    dz = d_ref[:, 2:3]
    tx = dx - sx
    ty = dy - sy
    tz = dz - sz
    d2 = tx * tx + ty * ty + tz * tz
    mind = jnp.min(d2, axis=0, keepdims=True)
    iota0 = lax.broadcasted_iota(jnp.int32, d2.shape, 0)
    bidx = jnp.min(jnp.where(d2 <= mind, iota0, _N), axis=0, keepdims=True)
    bidx_ref[0, :, :] = bidx
    dist_ref[0, :, :] = jnp.sqrt(jnp.maximum(mind, 0.0))


def _nn_tc(sx, sy, sz, dstC):
    nblk = _N // _BLK
    svec = pl.BlockSpec((_BLK,), lambda i: (i,))
    bidx, dist = pl.pallas_call(
        _nn_body,
        grid=(nblk,),
        in_specs=[svec, svec, svec, pl.BlockSpec((_N, 3), lambda i: (0, 0))],
        out_specs=[
            pl.BlockSpec((1, 1, _BLK), lambda i: (i, 0, 0)),
            pl.BlockSpec((1, 1, _BLK), lambda i: (i, 0, 0)),
        ],
        out_shape=[
            jax.ShapeDtypeStruct((nblk, 1, _BLK), jnp.int32),
            jax.ShapeDtypeStruct((nblk, 1, _BLK), _F32),
        ],
        interpret=_INTERPRET,
    )(sx, sy, sz, dstC)
    return bidx.reshape(-1), dist.reshape(-1)


def _lane_iota():
    return lax.iota(jnp.int32, 16)


def _extract_lane(v, k):
    return jnp.sum(jnp.where(_lane_iota() == k, v, jnp.zeros((16,), v.dtype)))


def _rsqrt_scalar(x):
    xv = jnp.full((16,), x, dtype=_F32)
    iv = plsc.bitcast(xv, jnp.int32)
    iv = 0x5F3759DF - lax.shift_right_logical(iv, 1)
    y = plsc.bitcast(iv, _F32)
    half = jnp.full((16,), 0.5, _F32) * xv
    for _ in range(3):
        y = y * (1.5 - half * y * y)
    return _extract_lane(y, 0)


def _jacobi_quat_fit(M, cA, cB):
    Sxx, Sxy, Sxz = M[0][0], M[0][1], M[0][2]
    Syx, Syy, Syz = M[1][0], M[1][1], M[1][2]
    Szx, Szy, Szz = M[2][0], M[2][1], M[2][2]
    N0 = [
        [Sxx + Syy + Szz, Syz - Szy, Szx - Sxz, Sxy - Syx],
        [Syz - Szy, Sxx - Syy - Szz, Sxy + Syx, Szx + Sxz],
        [Szx - Sxz, Sxy + Syx, -Sxx + Syy - Szz, Syz + Szy],
        [Sxy - Syx, Szx + Sxz, Syz + Szy, -Sxx - Syy + Szz],
    ]
    V0 = [[jnp.float32(1.0) if i == j else jnp.float32(0.0) for j in range(4)]
          for i in range(4)]

    def sweep(_, carry):
        flat = list(carry)
        Nk = [flat[4 * i:4 * i + 4] for i in range(4)]
        Vk = [flat[16 + 4 * i:16 + 4 * i + 4] for i in range(4)]
        for (p, q) in ((0, 1), (0, 2), (0, 3), (1, 2), (1, 3), (2, 3)):
            apq = Nk[p][q]
            d = Nk[q][q] - Nk[p][p]
            sgn = jnp.where(d >= 0.0, jnp.float32(1.0), jnp.float32(-1.0))
            rad = d * d + 4.0 * apq * apq
            root = jnp.where(rad > 0.0, rad * _rsqrt_scalar(rad + 1e-37), 0.0)
            den = jnp.abs(d) + root
            rden = _rsqrt_scalar(den + 1e-37)
            t = jnp.where(jnp.abs(apq) > 0.0,
                          (2.0 * apq * sgn) * (rden * rden), jnp.float32(0.0))
            c = _rsqrt_scalar(1.0 + t * t)
            s = t * c
            for k in range(4):
                nkp, nkq = Nk[k][p], Nk[k][q]
                Nk[k][p] = c * nkp - s * nkq
                Nk[k][q] = s * nkp + c * nkq
            for k in range(4):
                nkp, nkq = Nk[p][k], Nk[q][k]
                Nk[p][k] = c * nkp - s * nkq
                Nk[q][k] = s * nkp + c * nkq
            for k in range(4):
                vkp, vkq = Vk[k][p], Vk[k][q]
                Vk[k][p] = c * vkp - s * vkq
                Vk[k][q] = s * vkp + c * vkq
        return tuple(x for row in Nk for x in row) + \
               tuple(x for row in Vk for x in row)

    init = tuple(x for row in N0 for x in row) + \
           tuple(x for row in V0 for x in row)
    fin = lax.fori_loop(0, 6, sweep, init)
    Nd = [fin[0], fin[5], fin[10], fin[15]]
    Vf = [fin[16 + 4 * i:16 + 4 * i + 4] for i in range(4)]
    bl, bw, bx, by, bz = Nd[0], Vf[0][0], Vf[1][0], Vf[2][0], Vf[3][0]
    for k in (1, 2, 3):
        better = Nd[k] > bl
        bl = jnp.where(better, Nd[k], bl)
        bw = jnp.where(better, Vf[0][k], bw)
        bx = jnp.where(better, Vf[1][k], bx)
        by = jnp.where(better, Vf[2][k], by)
        bz = jnp.where(better, Vf[3][k], bz)
    w, x, y, z = bw, bx, by, bz
    R = [
        [w * w + x * x - y * y - z * z, 2 * (x * y - w * z), 2 * (x * z + w * y)],
        [2 * (x * y + w * z), w * w - x * x + y * y - z * z, 2 * (y * z - w * x)],
        [2 * (x * z - w * y), 2 * (y * z + w * x), w * w - x * x - y * y + z * z],
    ]
    t = [cB[j] - (R[j][0] * cA[0] + R[j][1] * cA[1] + R[j][2] * cA[2])
         for j in range(3)]
    return R, t


def _sc_fit_body(sx_hbm, sy_hbm, sz_hbm, dx_hbm, dy_hbm, dz_hbm,
                 bidx_hbm, dist_hbm,
                 ox_hbm, oy_hbm, oz_hbm, stats_hbm,
                 dxv, dyv, dzv, sxv, syv, szv, biv, dsv,
                 pvec, shared, allp, oxv, oyv, ozv):
    sid = lax.axis_index("s")
    base = sid * _RS

    pltpu.sync_copy(dx_hbm, dxv)
    pltpu.sync_copy(dy_hbm, dyv)
    pltpu.sync_copy(dz_hbm, dzv)
    pltpu.sync_copy(sx_hbm.at[pl.ds(base, _RS)], sxv)
    pltpu.sync_copy(sy_hbm.at[pl.ds(base, _RS)], syv)
    pltpu.sync_copy(sz_hbm.at[pl.ds(base, _RS)], szv)
    pltpu.sync_copy(bidx_hbm.at[pl.ds(base, _RS)], biv)
    pltpu.sync_copy(dist_hbm.at[pl.ds(base, _RS)], dsv)

    zero = jnp.zeros((16,), _F32)
    accs = [zero] * 16
    for c in range(_RS // 16):
        sl = pl.ds(c * 16, 16)
        sx = sxv[sl]
        sy = syv[sl]
        sz = szv[sl]
        dv = dsv[sl]
        ix = biv[sl]
        gx = plsc.load_gather(dxv, [ix])
        gy = plsc.load_gather(dyv, [ix])
        gz = plsc.load_gather(dzv, [ix])
        accs = [
            accs[0] + dv,
            accs[1] + sx, accs[2] + sy, accs[3] + sz,
            accs[4] + gx, accs[5] + gy, accs[6] + gz,
            accs[7] + sx * gx, accs[8] + sx * gy, accs[9] + sx * gz,
            accs[10] + sy * gx, accs[11] + sy * gy, accs[12] + sy * gz,
            accs[13] + sz * gx, accs[14] + sz * gy, accs[15] + sz * gz,
        ]
    lanes = _lane_iota()
    part = jnp.zeros((16,), _F32)
    for k in range(16):
        part = jnp.where(lanes == k, jnp.full((16,), jnp.sum(accs[k]), _F32),
                         part)
    pvec[...] = part
    pltpu.sync_copy(pvec, shared.at[pl.ds(sid * 16, 16)])
    plsc.subcore_barrier()
    pltpu.sync_copy(shared, allp)

    tot = allp[pl.ds(0, 16)]
    for k in range(1, _NSUB):
        tot = tot + allp[pl.ds(k * 16, 16)]

    inv_n = jnp.float32(1.0 / _N)
    sv = [_extract_lane(tot, k) for k in range(16)]
    sum_dist = sv[0]
    ss = sv[1:4]
    sg = sv[4:7]
    h = sv[7:16]
    cA = [ss[j] * inv_n for j in range(3)]
    cB = [sg[j] * inv_n for j in range(3)]
    M = [[h[3 * j + k] - ss[j] * sg[k] * inv_n for k in range(3)]
         for j in range(3)]
    R, t = _jacobi_quat_fit(M, cA, cB)
    mean_error = sum_dist * inv_n

    Rv = [[jnp.full((16,), R[j][k], _F32) for k in range(3)] for j in range(3)]
    tv = [jnp.full((16,), t[j], _F32) for j in range(3)]
    for c in range(_RS // 16):
        sl = pl.ds(c * 16, 16)
        sx = sxv[sl]
        sy = syv[sl]
        sz = szv[sl]
        oxv[sl] = Rv[0][0] * sx + Rv[0][1] * sy + Rv[0][2] * sz + tv[0]
        oyv[sl] = Rv[1][0] * sx + Rv[1][1] * sy + Rv[1][2] * sz + tv[1]
        ozv[sl] = Rv[2][0] * sx + Rv[2][1] * sy + Rv[2][2] * sz + tv[2]
    pltpu.sync_copy(oxv, ox_hbm.at[pl.ds(base, _RS)])
    pltpu.sync_copy(oyv, oy_hbm.at[pl.ds(base, _RS)])
    pltpu.sync_copy(ozv, oz_hbm.at[pl.ds(base, _RS)])

    flat = [mean_error] + [R[j][k] for j in range(3) for k in range(3)] + list(t)
    out = jnp.zeros((16,), _F32)
    for k in range(13):
        out = jnp.where(lanes == k, jnp.full((16,), flat[k], _F32), out)

    @pl.when(sid == 0)
    def _():
        pvec[...] = out
        pltpu.sync_copy(pvec, stats_hbm)


def _sc_fit_call(sx, sy, sz, dx, dy, dz, bidx, dist):
    mesh = plsc.VectorSubcoreMesh(core_axis_name="c", subcore_axis_name="s",
                                  num_cores=1, num_subcores=_NSUB)
    f = pl.kernel(
        _sc_fit_body,
        out_type=[
            jax.ShapeDtypeStruct((_N,), _F32),
            jax.ShapeDtypeStruct((_N,), _F32),
            jax.ShapeDtypeStruct((_N,), _F32),
            jax.ShapeDtypeStruct((16,), _F32),
        ],
        mesh=mesh,
        scratch_types=[
            pltpu.VMEM((_N,), _F32),
            pltpu.VMEM((_N,), _F32),
            pltpu.VMEM((_N,), _F32),
            pltpu.VMEM((_RS,), _F32),
            pltpu.VMEM((_RS,), _F32),
            pltpu.VMEM((_RS,), _F32),
            pltpu.VMEM((_RS,), jnp.int32),
            pltpu.VMEM((_RS,), _F32),
            pltpu.VMEM((16,), _F32),
            pltpu.VMEM_SHARED((_NSUB * 16,), _F32),
            pltpu.VMEM((_NSUB * 16,), _F32),
            pltpu.VMEM((_RS,), _F32),
            pltpu.VMEM((_RS,), _F32),
            pltpu.VMEM((_RS,), _F32),
        ],
        compiler_params=pltpu.CompilerParams(needs_layout_passes=False),
        interpret=_INTERPRET,
    )
    return f(sx, sy, sz, dx, dy, dz, bidx, dist)


def kernel(A, B):
    max_iterations = 20
    tolerance = 0.001
    dstC = B
    dx = B[:, 0]
    dy = B[:, 1]
    dz = B[:, 2]

    def cond(c):
        _, _, _, _, done, i = c
        return jnp.logical_and(i < max_iterations, jnp.logical_not(done))

    def body(c):
        sx, sy, sz, prev_error, done, i = c
        bidx, dist = _nn_tc(sx, sy, sz, dstC)
        nx, ny, nz, stats = _sc_fit_call(sx, sy, sz, dx, dy, dz, bidx, dist)
        mean_error = stats[0]
        converged = jnp.abs(prev_error - mean_error) < tolerance
        return (nx, ny, nz, mean_error, done | converged, i + 1)

    init = (A[:, 0], A[:, 1], A[:, 2], jnp.zeros((), A.dtype),
            jnp.array(False), jnp.array(0, jnp.int32))
    fx, fy, fz, _, _, _ = lax.while_loop(cond, body, init)

    ident = jnp.arange(_N, dtype=jnp.int32)
    zeros = jnp.zeros((_N,), _F32)
    _, _, _, stats = _sc_fit_call(A[:, 0], A[:, 1], A[:, 2], fx, fy, fz,
                                  ident, zeros)
    R = stats[1:10].reshape(3, 3)
    t = stats[10:13]
    T = jnp.eye(4, dtype=A.dtype)
    T = T.at[:3, :3].set(R)
    T = T.at[:3, 3].set(t)
    return T


# speedup vs baseline: 7.0954x; 1.3040x over previous
import jax
import jax.numpy as jnp
from jax import lax
from jax.experimental import pallas as pl
from jax.experimental.pallas import tpu as pltpu
from jax.experimental.pallas import tpu_sc as plsc

_INTERPRET = False

_N = 4096
_BLK = 256
_NSUB = 16
_RS = _N // _NSUB
_F32 = jnp.float32


def _nn_body(sx_ref, sy_ref, sz_ref, d_ref, bidx_ref, dist_ref):
    sx = sx_ref[...][None, :]
    sy = sy_ref[...][None, :]
    sz = sz_ref[...][None, :]
    rhs = jnp.concatenate(
        [-2.0 * sx, -2.0 * sy, -2.0 * sz, jnp.ones_like(sx)], axis=0)
    f = jnp.dot(d_ref[...], rhs, preferred_element_type=jnp.float32)
    minf = jnp.min(f, axis=0, keepdims=True)
    iota0 = lax.broadcasted_iota(jnp.int32, f.shape, 0)
    bidx = jnp.min(jnp.where(f <= minf, iota0, _N), axis=0, keepdims=True)
    s2 = sx * sx + sy * sy + sz * sz
    bidx_ref[0, :, :] = bidx
    dist_ref[0, :, :] = jnp.sqrt(jnp.maximum(minf + s2, 0.0))


def _nn_tc(sx, sy, sz, dstP):
    nblk = _N // _BLK
    svec = pl.BlockSpec((_BLK,), lambda i: (i,))
    bidx, dist = pl.pallas_call(
        _nn_body,
        grid=(nblk,),
        in_specs=[svec, svec, svec, pl.BlockSpec((_N, 4), lambda i: (0, 0))],
        out_specs=[
            pl.BlockSpec((1, 1, _BLK), lambda i: (i, 0, 0)),
            pl.BlockSpec((1, 1, _BLK), lambda i: (i, 0, 0)),
        ],
        out_shape=[
            jax.ShapeDtypeStruct((nblk, 1, _BLK), jnp.int32),
            jax.ShapeDtypeStruct((nblk, 1, _BLK), _F32),
        ],
        interpret=_INTERPRET,
    )(sx, sy, sz, dstP)
    return bidx.reshape(-1), dist.reshape(-1)


def _lane_iota():
    return lax.iota(jnp.int32, 16)


def _extract_lane(v, k):
    return jnp.sum(jnp.where(_lane_iota() == k, v, jnp.zeros((16,), v.dtype)))


def _rsqrt_scalar(x):
    xv = jnp.full((16,), x, dtype=_F32)
    iv = plsc.bitcast(xv, jnp.int32)
    iv = 0x5F3759DF - lax.shift_right_logical(iv, 1)
    y = plsc.bitcast(iv, _F32)
    half = jnp.full((16,), 0.5, _F32) * xv
    for _ in range(3):
        y = y * (1.5 - half * y * y)
    return _extract_lane(y, 0)


def _jacobi_quat_fit(M, cA, cB):
    Sxx, Sxy, Sxz = M[0][0], M[0][1], M[0][2]
    Syx, Syy, Syz = M[1][0], M[1][1], M[1][2]
    Szx, Szy, Szz = M[2][0], M[2][1], M[2][2]
    N0 = [
        [Sxx + Syy + Szz, Syz - Szy, Szx - Sxz, Sxy - Syx],
        [Syz - Szy, Sxx - Syy - Szz, Sxy + Syx, Szx + Sxz],
        [Szx - Sxz, Sxy + Syx, -Sxx + Syy - Szz, Syz + Szy],
        [Sxy - Syx, Szx + Sxz, Syz + Szy, -Sxx - Syy + Szz],
    ]
    V0 = [[jnp.float32(1.0) if i == j else jnp.float32(0.0) for j in range(4)]
          for i in range(4)]

    def sweep(_, carry):
        flat = list(carry)
        Nk = [flat[4 * i:4 * i + 4] for i in range(4)]
        Vk = [flat[16 + 4 * i:16 + 4 * i + 4] for i in range(4)]
        for (p, q) in ((0, 1), (0, 2), (0, 3), (1, 2), (1, 3), (2, 3)):
            apq = Nk[p][q]
            d = Nk[q][q] - Nk[p][p]
            sgn = jnp.where(d >= 0.0, jnp.float32(1.0), jnp.float32(-1.0))
            rad = d * d + 4.0 * apq * apq
            root = jnp.where(rad > 0.0, rad * _rsqrt_scalar(rad + 1e-37), 0.0)
            den = jnp.abs(d) + root
            rden = _rsqrt_scalar(den + 1e-37)
            t = jnp.where(jnp.abs(apq) > 0.0,
                          (2.0 * apq * sgn) * (rden * rden), jnp.float32(0.0))
            c = _rsqrt_scalar(1.0 + t * t)
            s = t * c
            for k in range(4):
                nkp, nkq = Nk[k][p], Nk[k][q]
                Nk[k][p] = c * nkp - s * nkq
                Nk[k][q] = s * nkp + c * nkq
            for k in range(4):
                nkp, nkq = Nk[p][k], Nk[q][k]
                Nk[p][k] = c * nkp - s * nkq
                Nk[q][k] = s * nkp + c * nkq
            for k in range(4):
                vkp, vkq = Vk[k][p], Vk[k][q]
                Vk[k][p] = c * vkp - s * vkq
                Vk[k][q] = s * vkp + c * vkq
        return tuple(x for row in Nk for x in row) + \
               tuple(x for row in Vk for x in row)

    init = tuple(x for row in N0 for x in row) + \
           tuple(x for row in V0 for x in row)
    fin = lax.fori_loop(0, 6, sweep, init)
    Nd = [fin[0], fin[5], fin[10], fin[15]]
    Vf = [fin[16 + 4 * i:16 + 4 * i + 4] for i in range(4)]
    bl, bw, bx, by, bz = Nd[0], Vf[0][0], Vf[1][0], Vf[2][0], Vf[3][0]
    for k in (1, 2, 3):
        better = Nd[k] > bl
        bl = jnp.where(better, Nd[k], bl)
        bw = jnp.where(better, Vf[0][k], bw)
        bx = jnp.where(better, Vf[1][k], bx)
        by = jnp.where(better, Vf[2][k], by)
        bz = jnp.where(better, Vf[3][k], bz)
    w, x, y, z = bw, bx, by, bz
    R = [
        [w * w + x * x - y * y - z * z, 2 * (x * y - w * z), 2 * (x * z + w * y)],
        [2 * (x * y + w * z), w * w - x * x + y * y - z * z, 2 * (y * z - w * x)],
        [2 * (x * z - w * y), 2 * (y * z + w * x), w * w - x * x - y * y + z * z],
    ]
    t = [cB[j] - (R[j][0] * cA[0] + R[j][1] * cA[1] + R[j][2] * cA[2])
         for j in range(3)]
    return R, t


def _sc_fit_body(sx_hbm, sy_hbm, sz_hbm, dx_hbm, dy_hbm, dz_hbm,
                 bidx_hbm, dist_hbm,
                 ox_hbm, oy_hbm, oz_hbm, stats_hbm,
                 dxv, dyv, dzv, sxv, syv, szv, biv, dsv,
                 pvec, shared, allp, oxv, oyv, ozv):
    sid = lax.axis_index("s")
    base = sid * _RS

    pltpu.sync_copy(dx_hbm, dxv)
    pltpu.sync_copy(dy_hbm, dyv)
    pltpu.sync_copy(dz_hbm, dzv)
    pltpu.sync_copy(sx_hbm.at[pl.ds(base, _RS)], sxv)
    pltpu.sync_copy(sy_hbm.at[pl.ds(base, _RS)], syv)
    pltpu.sync_copy(sz_hbm.at[pl.ds(base, _RS)], szv)
    pltpu.sync_copy(bidx_hbm.at[pl.ds(base, _RS)], biv)
    pltpu.sync_copy(dist_hbm.at[pl.ds(base, _RS)], dsv)

    zero = jnp.zeros((16,), _F32)
    accs = [zero] * 16
    for c in range(_RS // 16):
        sl = pl.ds(c * 16, 16)
        sx = sxv[sl]
        sy = syv[sl]
        sz = szv[sl]
        dv = dsv[sl]
        ix = biv[sl]
        gx = plsc.load_gather(dxv, [ix])
        gy = plsc.load_gather(dyv, [ix])
        gz = plsc.load_gather(dzv, [ix])
        accs = [
            accs[0] + dv,
            accs[1] + sx, accs[2] + sy, accs[3] + sz,
            accs[4] + gx, accs[5] + gy, accs[6] + gz,
            accs[7] + sx * gx, accs[8] + sx * gy, accs[9] + sx * gz,
            accs[10] + sy * gx, accs[11] + sy * gy, accs[12] + sy * gz,
            accs[13] + sz * gx, accs[14] + sz * gy, accs[15] + sz * gz,
        ]
    lanes = _lane_iota()
    part = jnp.zeros((16,), _F32)
    for k in range(16):
        part = jnp.where(lanes == k, jnp.full((16,), jnp.sum(accs[k]), _F32),
                         part)
    pvec[...] = part
    pltpu.sync_copy(pvec, shared.at[pl.ds(sid * 16, 16)])
    plsc.subcore_barrier()
    pltpu.sync_copy(shared, allp)

    tot = allp[pl.ds(0, 16)]
    for k in range(1, _NSUB):
        tot = tot + allp[pl.ds(k * 16, 16)]

    inv_n = jnp.float32(1.0 / _N)
    sv = [_extract_lane(tot, k) for k in range(16)]
    sum_dist = sv[0]
    ss = sv[1:4]
    sg = sv[4:7]
    h = sv[7:16]
    cA = [ss[j] * inv_n for j in range(3)]
    cB = [sg[j] * inv_n for j in range(3)]
    M = [[h[3 * j + k] - ss[j] * sg[k] * inv_n for k in range(3)]
         for j in range(3)]
    R, t = _jacobi_quat_fit(M, cA, cB)
    mean_error = sum_dist * inv_n

    Rv = [[jnp.full((16,), R[j][k], _F32) for k in range(3)] for j in range(3)]
    tv = [jnp.full((16,), t[j], _F32) for j in range(3)]
    for c in range(_RS // 16):
        sl = pl.ds(c * 16, 16)
        sx = sxv[sl]
        sy = syv[sl]
        sz = szv[sl]
        oxv[sl] = Rv[0][0] * sx + Rv[0][1] * sy + Rv[0][2] * sz + tv[0]
        oyv[sl] = Rv[1][0] * sx + Rv[1][1] * sy + Rv[1][2] * sz + tv[1]
        ozv[sl] = Rv[2][0] * sx + Rv[2][1] * sy + Rv[2][2] * sz + tv[2]
    pltpu.sync_copy(oxv, ox_hbm.at[pl.ds(base, _RS)])
    pltpu.sync_copy(oyv, oy_hbm.at[pl.ds(base, _RS)])
    pltpu.sync_copy(ozv, oz_hbm.at[pl.ds(base, _RS)])

    flat = [mean_error] + [R[j][k] for j in range(3) for k in range(3)] + list(t)
    out = jnp.zeros((16,), _F32)
    for k in range(13):
        out = jnp.where(lanes == k, jnp.full((16,), flat[k], _F32), out)

    @pl.when(sid == 0)
    def _():
        pvec[...] = out
        pltpu.sync_copy(pvec, stats_hbm)


def _sc_fit_call(sx, sy, sz, dx, dy, dz, bidx, dist):
    mesh = plsc.VectorSubcoreMesh(core_axis_name="c", subcore_axis_name="s",
                                  num_cores=1, num_subcores=_NSUB)
    f = pl.kernel(
        _sc_fit_body,
        out_type=[
            jax.ShapeDtypeStruct((_N,), _F32),
            jax.ShapeDtypeStruct((_N,), _F32),
            jax.ShapeDtypeStruct((_N,), _F32),
            jax.ShapeDtypeStruct((16,), _F32),
        ],
        mesh=mesh,
        scratch_types=[
            pltpu.VMEM((_N,), _F32),
            pltpu.VMEM((_N,), _F32),
            pltpu.VMEM((_N,), _F32),
            pltpu.VMEM((_RS,), _F32),
            pltpu.VMEM((_RS,), _F32),
            pltpu.VMEM((_RS,), _F32),
            pltpu.VMEM((_RS,), jnp.int32),
            pltpu.VMEM((_RS,), _F32),
            pltpu.VMEM((16,), _F32),
            pltpu.VMEM_SHARED((_NSUB * 16,), _F32),
            pltpu.VMEM((_NSUB * 16,), _F32),
            pltpu.VMEM((_RS,), _F32),
            pltpu.VMEM((_RS,), _F32),
            pltpu.VMEM((_RS,), _F32),
        ],
        compiler_params=pltpu.CompilerParams(needs_layout_passes=False),
        interpret=_INTERPRET,
    )
    return f(sx, sy, sz, dx, dy, dz, bidx, dist)


def kernel(A, B):
    max_iterations = 20
    tolerance = 0.001
    dx = B[:, 0]
    dy = B[:, 1]
    dz = B[:, 2]
    qd = dx * dx + dy * dy + dz * dz
    dstP = jnp.concatenate([B, qd[:, None]], axis=1)

    def cond(c):
        _, _, _, _, done, i = c
        return jnp.logical_and(i < max_iterations, jnp.logical_not(done))

    def body(c):
        sx, sy, sz, prev_error, done, i = c
        bidx, dist = _nn_tc(sx, sy, sz, dstP)
        nx, ny, nz, stats = _sc_fit_call(sx, sy, sz, dx, dy, dz, bidx, dist)
        mean_error = stats[0]
        converged = jnp.abs(prev_error - mean_error) < tolerance
        return (nx, ny, nz, mean_error, done | converged, i + 1)

    init = (A[:, 0], A[:, 1], A[:, 2], jnp.zeros((), A.dtype),
            jnp.array(False), jnp.array(0, jnp.int32))
    fx, fy, fz, _, _, _ = lax.while_loop(cond, body, init)

    ident = jnp.arange(_N, dtype=jnp.int32)
    zeros = jnp.zeros((_N,), _F32)
    _, _, _, stats = _sc_fit_call(A[:, 0], A[:, 1], A[:, 2], fx, fy, fz,
                                  ident, zeros)
    R = stats[1:10].reshape(3, 3)
    t = stats[10:13]
    T = jnp.eye(4, dtype=A.dtype)
    T = T.at[:3, :3].set(R)
    T = T.at[:3, 3].set(t)
    return T


# speedup vs baseline: 8.2203x; 1.1585x over previous
import jax
import jax.numpy as jnp
from jax import lax
from jax.experimental import pallas as pl
from jax.experimental.pallas import tpu as pltpu
from jax.experimental.pallas import tpu_sc as plsc

_INTERPRET = False

_N = 4096
_BLK = 256
_NSUB = 16
_RS = _N // _NSUB
_F32 = jnp.float32


def _nn_body(sx_ref, sy_ref, sz_ref, d_ref, bidx_ref, dist_ref):
    sx = sx_ref[...][None, :]
    sy = sy_ref[...][None, :]
    sz = sz_ref[...][None, :]
    rhs = jnp.concatenate(
        [-2.0 * sx, -2.0 * sy, -2.0 * sz, jnp.ones_like(sx)], axis=0)
    f = jnp.dot(d_ref[...], rhs, preferred_element_type=jnp.float32)
    minf = jnp.min(f, axis=0, keepdims=True)
    iota0 = lax.broadcasted_iota(jnp.int32, f.shape, 0)
    bidx = jnp.min(jnp.where(f <= minf, iota0, _N), axis=0, keepdims=True)
    s2 = sx * sx + sy * sy + sz * sz
    bidx_ref[0, :, :] = bidx
    dist_ref[0, :, :] = jnp.sqrt(jnp.maximum(minf + s2, 0.0))


def _nn_tc(sx, sy, sz, dstP):
    nblk = _N // _BLK
    svec = pl.BlockSpec((_BLK,), lambda i: (i,))
    bidx, dist = pl.pallas_call(
        _nn_body,
        grid=(nblk,),
        in_specs=[svec, svec, svec, pl.BlockSpec((_N, 4), lambda i: (0, 0))],
        out_specs=[
            pl.BlockSpec((1, 1, _BLK), lambda i: (i, 0, 0)),
            pl.BlockSpec((1, 1, _BLK), lambda i: (i, 0, 0)),
        ],
        out_shape=[
            jax.ShapeDtypeStruct((nblk, 1, _BLK), jnp.int32),
            jax.ShapeDtypeStruct((nblk, 1, _BLK), _F32),
        ],
        interpret=_INTERPRET,
    )(sx, sy, sz, dstP)
    return bidx.reshape(-1), dist.reshape(-1)


def _lane_iota():
    return lax.iota(jnp.int32, 16)


def _extract_lane(v, k):
    return jnp.sum(jnp.where(_lane_iota() == k, v, jnp.zeros((16,), v.dtype)))


def _rsqrt_scalar(x):
    xv = jnp.full((16,), x, dtype=_F32)
    iv = plsc.bitcast(xv, jnp.int32)
    iv = 0x5F3759DF - lax.shift_right_logical(iv, 1)
    y = plsc.bitcast(iv, _F32)
    half = jnp.full((16,), 0.5, _F32) * xv
    for _ in range(3):
        y = y * (1.5 - half * y * y)
    return _extract_lane(y, 0)


def _jacobi_quat_fit(M, cA, cB):
    Sxx, Sxy, Sxz = M[0][0], M[0][1], M[0][2]
    Syx, Syy, Syz = M[1][0], M[1][1], M[1][2]
    Szx, Szy, Szz = M[2][0], M[2][1], M[2][2]
    N0 = [
        [Sxx + Syy + Szz, Syz - Szy, Szx - Sxz, Sxy - Syx],
        [Syz - Szy, Sxx - Syy - Szz, Sxy + Syx, Szx + Sxz],
        [Szx - Sxz, Sxy + Syx, -Sxx + Syy - Szz, Syz + Szy],
        [Sxy - Syx, Szx + Sxz, Syz + Szy, -Sxx - Syy + Szz],
    ]
    V0 = [[jnp.float32(1.0) if i == j else jnp.float32(0.0) for j in range(4)]
          for i in range(4)]

    def sweep(_, carry):
        flat = list(carry)
        Nk = [flat[4 * i:4 * i + 4] for i in range(4)]
        Vk = [flat[16 + 4 * i:16 + 4 * i + 4] for i in range(4)]
        for (p, q) in ((0, 1), (0, 2), (0, 3), (1, 2), (1, 3), (2, 3)):
            apq = Nk[p][q]
            d = Nk[q][q] - Nk[p][p]
            sgn = jnp.where(d >= 0.0, jnp.float32(1.0), jnp.float32(-1.0))
            rad = d * d + 4.0 * apq * apq
            root = jnp.where(rad > 0.0, rad * _rsqrt_scalar(rad + 1e-37), 0.0)
            den = jnp.abs(d) + root
            rden = _rsqrt_scalar(den + 1e-37)
            t = jnp.where(jnp.abs(apq) > 0.0,
                          (2.0 * apq * sgn) * (rden * rden), jnp.float32(0.0))
            c = _rsqrt_scalar(1.0 + t * t)
            s = t * c
            for k in range(4):
                nkp, nkq = Nk[k][p], Nk[k][q]
                Nk[k][p] = c * nkp - s * nkq
                Nk[k][q] = s * nkp + c * nkq
            for k in range(4):
                nkp, nkq = Nk[p][k], Nk[q][k]
                Nk[p][k] = c * nkp - s * nkq
                Nk[q][k] = s * nkp + c * nkq
            for k in range(4):
                vkp, vkq = Vk[k][p], Vk[k][q]
                Vk[k][p] = c * vkp - s * vkq
                Vk[k][q] = s * vkp + c * vkq
        return tuple(x for row in Nk for x in row) + \
               tuple(x for row in Vk for x in row)

    init = tuple(x for row in N0 for x in row) + \
           tuple(x for row in V0 for x in row)
    fin = lax.fori_loop(0, 5, sweep, init)
    Nd = [fin[0], fin[5], fin[10], fin[15]]
    Vf = [fin[16 + 4 * i:16 + 4 * i + 4] for i in range(4)]
    bl, bw, bx, by, bz = Nd[0], Vf[0][0], Vf[1][0], Vf[2][0], Vf[3][0]
    for k in (1, 2, 3):
        better = Nd[k] > bl
        bl = jnp.where(better, Nd[k], bl)
        bw = jnp.where(better, Vf[0][k], bw)
        bx = jnp.where(better, Vf[1][k], bx)
        by = jnp.where(better, Vf[2][k], by)
        bz = jnp.where(better, Vf[3][k], bz)
    w, x, y, z = bw, bx, by, bz
    R = [
        [w * w + x * x - y * y - z * z, 2 * (x * y - w * z), 2 * (x * z + w * y)],
        [2 * (x * y + w * z), w * w - x * x + y * y - z * z, 2 * (y * z - w * x)],
        [2 * (x * z - w * y), 2 * (y * z + w * x), w * w - x * x - y * y + z * z],
    ]
    t = [cB[j] - (R[j][0] * cA[0] + R[j][1] * cA[1] + R[j][2] * cA[2])
         for j in range(3)]
    return R, t


def _sc_fit_body(sx_hbm, sy_hbm, sz_hbm, dx_hbm, dy_hbm, dz_hbm,
                 bidx_hbm, dist_hbm, pstat_hbm,
                 ox_hbm, oy_hbm, oz_hbm, stats_hbm,
                 dxv, dyv, dzv, sxv, syv, szv, biv, dsv,
                 pvec, shared, allp, oxv, oyv, ozv, pstatv):
    sid = lax.axis_index("s")
    base = sid * _RS

    pltpu.sync_copy(dx_hbm, dxv)
    pltpu.sync_copy(dy_hbm, dyv)
    pltpu.sync_copy(dz_hbm, dzv)
    pltpu.sync_copy(sx_hbm.at[pl.ds(base, _RS)], sxv)
    pltpu.sync_copy(sy_hbm.at[pl.ds(base, _RS)], syv)
    pltpu.sync_copy(sz_hbm.at[pl.ds(base, _RS)], szv)
    pltpu.sync_copy(bidx_hbm.at[pl.ds(base, _RS)], biv)
    pltpu.sync_copy(dist_hbm.at[pl.ds(base, _RS)], dsv)
    pltpu.sync_copy(pstat_hbm, pstatv)

    zero = jnp.zeros((16,), _F32)
    accs = [zero] * 16
    for c in range(_RS // 16):
        sl = pl.ds(c * 16, 16)
        sx = sxv[sl]
        sy = syv[sl]
        sz = szv[sl]
        dv = dsv[sl]
        ix = biv[sl]
        gx = plsc.load_gather(dxv, [ix])
        gy = plsc.load_gather(dyv, [ix])
        gz = plsc.load_gather(dzv, [ix])
        accs = [
            accs[0] + dv,
            accs[1] + sx, accs[2] + sy, accs[3] + sz,
            accs[4] + gx, accs[5] + gy, accs[6] + gz,
            accs[7] + sx * gx, accs[8] + sx * gy, accs[9] + sx * gz,
            accs[10] + sy * gx, accs[11] + sy * gy, accs[12] + sy * gz,
            accs[13] + sz * gx, accs[14] + sz * gy, accs[15] + sz * gz,
        ]
    lanes = _lane_iota()
    part = jnp.zeros((16,), _F32)
    for k in range(16):
        part = jnp.where(lanes == k, jnp.full((16,), jnp.sum(accs[k]), _F32),
                         part)
    pvec[...] = part
    pltpu.sync_copy(pvec, shared.at[pl.ds(sid * 16, 16)])
    plsc.subcore_barrier()
    pltpu.sync_copy(shared, allp)

    tot = allp[pl.ds(0, 16)]
    for k in range(1, _NSUB):
        tot = tot + allp[pl.ds(k * 16, 16)]

    inv_n = jnp.float32(1.0 / _N)
    sv = [_extract_lane(tot, k) for k in range(16)]
    sum_dist = sv[0]
    ss = sv[1:4]
    sg = sv[4:7]
    h = sv[7:16]
    cA = [ss[j] * inv_n for j in range(3)]
    cB = [sg[j] * inv_n for j in range(3)]
    M = [[h[3 * j + k] - ss[j] * sg[k] * inv_n for k in range(3)]
         for j in range(3)]
    R, t = _jacobi_quat_fit(M, cA, cB)
    mean_error = sum_dist * inv_n

    Rv = [[jnp.full((16,), R[j][k], _F32) for k in range(3)] for j in range(3)]
    tv = [jnp.full((16,), t[j], _F32) for j in range(3)]
    for c in range(_RS // 16):
        sl = pl.ds(c * 16, 16)
        sx = sxv[sl]
        sy = syv[sl]
        sz = szv[sl]
        oxv[sl] = Rv[0][0] * sx + Rv[0][1] * sy + Rv[0][2] * sz + tv[0]
        oyv[sl] = Rv[1][0] * sx + Rv[1][1] * sy + Rv[1][2] * sz + tv[1]
        ozv[sl] = Rv[2][0] * sx + Rv[2][1] * sy + Rv[2][2] * sz + tv[2]
    pltpu.sync_copy(oxv, ox_hbm.at[pl.ds(base, _RS)])
    pltpu.sync_copy(oyv, oy_hbm.at[pl.ds(base, _RS)])
    pltpu.sync_copy(ozv, oz_hbm.at[pl.ds(base, _RS)])

    ps = pstatv[...]
    Rp = [[_extract_lane(ps, 1 + 3 * j + k) for k in range(3)] for j in range(3)]
    tp = [_extract_lane(ps, 10 + j) for j in range(3)]
    Rn = [[R[j][0] * Rp[0][k] + R[j][1] * Rp[1][k] + R[j][2] * Rp[2][k]
           for k in range(3)] for j in range(3)]
    tn = [R[j][0] * tp[0] + R[j][1] * tp[1] + R[j][2] * tp[2] + t[j]
          for j in range(3)]

    flat = [mean_error] + [Rn[j][k] for j in range(3) for k in range(3)] + \
        list(tn)
    out = jnp.zeros((16,), _F32)
    for k in range(13):
        out = jnp.where(lanes == k, jnp.full((16,), flat[k], _F32), out)

    @pl.when(sid == 0)
    def _():
        pvec[...] = out
        pltpu.sync_copy(pvec, stats_hbm)


def _sc_fit_call(sx, sy, sz, dx, dy, dz, bidx, dist, pstat):
    mesh = plsc.VectorSubcoreMesh(core_axis_name="c", subcore_axis_name="s",
                                  num_cores=1, num_subcores=_NSUB)
    f = pl.kernel(
        _sc_fit_body,
        out_type=[
            jax.ShapeDtypeStruct((_N,), _F32),
            jax.ShapeDtypeStruct((_N,), _F32),
            jax.ShapeDtypeStruct((_N,), _F32),
            jax.ShapeDtypeStruct((16,), _F32),
        ],
        mesh=mesh,
        scratch_types=[
            pltpu.VMEM((_N,), _F32),
            pltpu.VMEM((_N,), _F32),
            pltpu.VMEM((_N,), _F32),
            pltpu.VMEM((_RS,), _F32),
            pltpu.VMEM((_RS,), _F32),
            pltpu.VMEM((_RS,), _F32),
            pltpu.VMEM((_RS,), jnp.int32),
            pltpu.VMEM((_RS,), _F32),
            pltpu.VMEM((16,), _F32),
            pltpu.VMEM_SHARED((_NSUB * 16,), _F32),
            pltpu.VMEM((_NSUB * 16,), _F32),
            pltpu.VMEM((_RS,), _F32),
            pltpu.VMEM((_RS,), _F32),
            pltpu.VMEM((_RS,), _F32),
            pltpu.VMEM((16,), _F32),
        ],
        compiler_params=pltpu.CompilerParams(needs_layout_passes=False),
        interpret=_INTERPRET,
    )
    return f(sx, sy, sz, dx, dy, dz, bidx, dist, pstat)


def kernel(A, B):
    max_iterations = 20
    tolerance = 0.001
    dx = B[:, 0]
    dy = B[:, 1]
    dz = B[:, 2]
    qd = dx * dx + dy * dy + dz * dz
    dstP = jnp.concatenate([B, qd[:, None]], axis=1)

    def cond(c):
        _, _, _, _, _, done, i = c
        return jnp.logical_and(i < max_iterations, jnp.logical_not(done))

    def body(c):
        sx, sy, sz, pstat, prev_error, done, i = c
        bidx, dist = _nn_tc(sx, sy, sz, dstP)
        nx, ny, nz, stats = _sc_fit_call(sx, sy, sz, dx, dy, dz, bidx, dist,
                                         pstat)
        mean_error = stats[0]
        converged = jnp.abs(prev_error - mean_error) < tolerance
        return (nx, ny, nz, stats, mean_error, done | converged, i + 1)

    stat0 = jnp.array([0, 1, 0, 0, 0, 1, 0, 0, 0, 1, 0, 0, 0, 0, 0, 0],
                      dtype=_F32)
    init = (A[:, 0], A[:, 1], A[:, 2], stat0, jnp.zeros((), A.dtype),
            jnp.array(False), jnp.array(0, jnp.int32))
    _, _, _, stats, _, _, _ = lax.while_loop(cond, body, init)

    R = stats[1:10].reshape(3, 3)
    t = stats[10:13]
    T = jnp.eye(4, dtype=A.dtype)
    T = T.at[:3, :3].set(R)
    T = T.at[:3, 3].set(t)
    return T


# speedup vs baseline: 8.5451x; 1.0395x over previous
import jax
import jax.numpy as jnp
from jax import lax
from jax.experimental import pallas as pl
from jax.experimental.pallas import tpu as pltpu
from jax.experimental.pallas import tpu_sc as plsc

_INTERPRET = False

_N = 4096
_BLK = 512
_NSUB = 16
_RS = _N // _NSUB
_F32 = jnp.float32


def _nn_body(sx_ref, sy_ref, sz_ref, d_ref, bidx_ref, dist_ref):
    sx = sx_ref[...][None, :]
    sy = sy_ref[...][None, :]
    sz = sz_ref[...][None, :]
    rhs = jnp.concatenate(
        [-2.0 * sx, -2.0 * sy, -2.0 * sz, jnp.ones_like(sx)], axis=0)
    f = jnp.dot(d_ref[...], rhs, preferred_element_type=jnp.float32)
    minf = jnp.min(f, axis=0, keepdims=True)
    iota0 = lax.broadcasted_iota(jnp.int32, f.shape, 0)
    bidx = jnp.min(jnp.where(f <= minf, iota0, _N), axis=0, keepdims=True)
    s2 = sx * sx + sy * sy + sz * sz
    bidx_ref[0, :, :] = bidx
    dist_ref[0, :, :] = jnp.sqrt(jnp.maximum(minf + s2, 0.0))


def _nn_tc(sx, sy, sz, dstP):
    nblk = _N // _BLK
    svec = pl.BlockSpec((_BLK,), lambda i: (i,))
    bidx, dist = pl.pallas_call(
        _nn_body,
        grid=(nblk,),
        in_specs=[svec, svec, svec, pl.BlockSpec((_N, 4), lambda i: (0, 0))],
        out_specs=[
            pl.BlockSpec((1, 1, _BLK), lambda i: (i, 0, 0)),
            pl.BlockSpec((1, 1, _BLK), lambda i: (i, 0, 0)),
        ],
        out_shape=[
            jax.ShapeDtypeStruct((nblk, 1, _BLK), jnp.int32),
            jax.ShapeDtypeStruct((nblk, 1, _BLK), _F32),
        ],
        interpret=_INTERPRET,
    )(sx, sy, sz, dstP)
    return bidx.reshape(-1), dist.reshape(-1)


def _lane_iota():
    return lax.iota(jnp.int32, 16)


def _extract_lane(v, k):
    return jnp.sum(jnp.where(_lane_iota() == k, v, jnp.zeros((16,), v.dtype)))


def _rsqrt_scalar(x):
    xv = jnp.full((16,), x, dtype=_F32)
    iv = plsc.bitcast(xv, jnp.int32)
    iv = 0x5F3759DF - lax.shift_right_logical(iv, 1)
    y = plsc.bitcast(iv, _F32)
    half = jnp.full((16,), 0.5, _F32) * xv
    for _ in range(3):
        y = y * (1.5 - half * y * y)
    return _extract_lane(y, 0)


def _jacobi_quat_fit(M, cA, cB):
    Sxx, Sxy, Sxz = M[0][0], M[0][1], M[0][2]
    Syx, Syy, Syz = M[1][0], M[1][1], M[1][2]
    Szx, Szy, Szz = M[2][0], M[2][1], M[2][2]
    N0 = [
        [Sxx + Syy + Szz, Syz - Szy, Szx - Sxz, Sxy - Syx],
        [Syz - Szy, Sxx - Syy - Szz, Sxy + Syx, Szx + Sxz],
        [Szx - Sxz, Sxy + Syx, -Sxx + Syy - Szz, Syz + Szy],
        [Sxy - Syx, Szx + Sxz, Syz + Szy, -Sxx - Syy + Szz],
    ]
    V0 = [[jnp.float32(1.0) if i == j else jnp.float32(0.0) for j in range(4)]
          for i in range(4)]

    def sweep(_, carry):
        flat = list(carry)
        Nk = [flat[4 * i:4 * i + 4] for i in range(4)]
        Vk = [flat[16 + 4 * i:16 + 4 * i + 4] for i in range(4)]
        for (p, q) in ((0, 1), (0, 2), (0, 3), (1, 2), (1, 3), (2, 3)):
            apq = Nk[p][q]
            d = Nk[q][q] - Nk[p][p]
            sgn = jnp.where(d >= 0.0, jnp.float32(1.0), jnp.float32(-1.0))
            rad = d * d + 4.0 * apq * apq
            root = jnp.where(rad > 0.0, rad * _rsqrt_scalar(rad + 1e-37), 0.0)
            den = jnp.abs(d) + root
            rden = _rsqrt_scalar(den + 1e-37)
            t = jnp.where(jnp.abs(apq) > 0.0,
                          (2.0 * apq * sgn) * (rden * rden), jnp.float32(0.0))
            c = _rsqrt_scalar(1.0 + t * t)
            s = t * c
            for k in range(4):
                nkp, nkq = Nk[k][p], Nk[k][q]
                Nk[k][p] = c * nkp - s * nkq
                Nk[k][q] = s * nkp + c * nkq
            for k in range(4):
                nkp, nkq = Nk[p][k], Nk[q][k]
                Nk[p][k] = c * nkp - s * nkq
                Nk[q][k] = s * nkp + c * nkq
            for k in range(4):
                vkp, vkq = Vk[k][p], Vk[k][q]
                Vk[k][p] = c * vkp - s * vkq
                Vk[k][q] = s * vkp + c * vkq
        return tuple(x for row in Nk for x in row) + \
               tuple(x for row in Vk for x in row)

    init = tuple(x for row in N0 for x in row) + \
           tuple(x for row in V0 for x in row)
    fin = lax.fori_loop(0, 5, sweep, init)
    Nd = [fin[0], fin[5], fin[10], fin[15]]
    Vf = [fin[16 + 4 * i:16 + 4 * i + 4] for i in range(4)]
    bl, bw, bx, by, bz = Nd[0], Vf[0][0], Vf[1][0], Vf[2][0], Vf[3][0]
    for k in (1, 2, 3):
        better = Nd[k] > bl
        bl = jnp.where(better, Nd[k], bl)
        bw = jnp.where(better, Vf[0][k], bw)
        bx = jnp.where(better, Vf[1][k], bx)
        by = jnp.where(better, Vf[2][k], by)
        bz = jnp.where(better, Vf[3][k], bz)
    w, x, y, z = bw, bx, by, bz
    R = [
        [w * w + x * x - y * y - z * z, 2 * (x * y - w * z), 2 * (x * z + w * y)],
        [2 * (x * y + w * z), w * w - x * x + y * y - z * z, 2 * (y * z - w * x)],
        [2 * (x * z - w * y), 2 * (y * z + w * x), w * w - x * x - y * y + z * z],
    ]
    t = [cB[j] - (R[j][0] * cA[0] + R[j][1] * cA[1] + R[j][2] * cA[2])
         for j in range(3)]
    return R, t


def _sc_fit_body(sx_hbm, sy_hbm, sz_hbm, dx_hbm, dy_hbm, dz_hbm,
                 bidx_hbm, dist_hbm, pstat_hbm,
                 ox_hbm, oy_hbm, oz_hbm, stats_hbm,
                 dxv, dyv, dzv, sxv, syv, szv, biv, dsv,
                 pvec, shared, allp, oxv, oyv, ozv, pstatv):
    sid = lax.axis_index("s")
    base = sid * _RS

    pltpu.sync_copy(dx_hbm, dxv)
    pltpu.sync_copy(dy_hbm, dyv)
    pltpu.sync_copy(dz_hbm, dzv)
    pltpu.sync_copy(sx_hbm.at[pl.ds(base, _RS)], sxv)
    pltpu.sync_copy(sy_hbm.at[pl.ds(base, _RS)], syv)
    pltpu.sync_copy(sz_hbm.at[pl.ds(base, _RS)], szv)
    pltpu.sync_copy(bidx_hbm.at[pl.ds(base, _RS)], biv)
    pltpu.sync_copy(dist_hbm.at[pl.ds(base, _RS)], dsv)
    pltpu.sync_copy(pstat_hbm, pstatv)

    zero = jnp.zeros((16,), _F32)
    accs = [zero] * 16
    for c in range(_RS // 16):
        sl = pl.ds(c * 16, 16)
        sx = sxv[sl]
        sy = syv[sl]
        sz = szv[sl]
        dv = dsv[sl]
        ix = biv[sl]
        gx = plsc.load_gather(dxv, [ix])
        gy = plsc.load_gather(dyv, [ix])
        gz = plsc.load_gather(dzv, [ix])
        accs = [
            accs[0] + dv,
            accs[1] + sx, accs[2] + sy, accs[3] + sz,
            accs[4] + gx, accs[5] + gy, accs[6] + gz,
            accs[7] + sx * gx, accs[8] + sx * gy, accs[9] + sx * gz,
            accs[10] + sy * gx, accs[11] + sy * gy, accs[12] + sy * gz,
            accs[13] + sz * gx, accs[14] + sz * gy, accs[15] + sz * gz,
        ]
    lanes = _lane_iota()
    part = jnp.zeros((16,), _F32)
    for k in range(16):
        part = jnp.where(lanes == k, jnp.full((16,), jnp.sum(accs[k]), _F32),
                         part)
    pvec[...] = part
    pltpu.sync_copy(pvec, shared.at[pl.ds(sid * 16, 16)])
    plsc.subcore_barrier()
    pltpu.sync_copy(shared, allp)

    tot = allp[pl.ds(0, 16)]
    for k in range(1, _NSUB):
        tot = tot + allp[pl.ds(k * 16, 16)]

    inv_n = jnp.float32(1.0 / _N)
    sv = [_extract_lane(tot, k) for k in range(16)]
    sum_dist = sv[0]
    ss = sv[1:4]
    sg = sv[4:7]
    h = sv[7:16]
    cA = [ss[j] * inv_n for j in range(3)]
    cB = [sg[j] * inv_n for j in range(3)]
    M = [[h[3 * j + k] - ss[j] * sg[k] * inv_n for k in range(3)]
         for j in range(3)]
    R, t = _jacobi_quat_fit(M, cA, cB)
    mean_error = sum_dist * inv_n

    Rv = [[jnp.full((16,), R[j][k], _F32) for k in range(3)] for j in range(3)]
    tv = [jnp.full((16,), t[j], _F32) for j in range(3)]
    for c in range(_RS // 16):
        sl = pl.ds(c * 16, 16)
        sx = sxv[sl]
        sy = syv[sl]
        sz = szv[sl]
        oxv[sl] = Rv[0][0] * sx + Rv[0][1] * sy + Rv[0][2] * sz + tv[0]
        oyv[sl] = Rv[1][0] * sx + Rv[1][1] * sy + Rv[1][2] * sz + tv[1]
        ozv[sl] = Rv[2][0] * sx + Rv[2][1] * sy + Rv[2][2] * sz + tv[2]
    pltpu.sync_copy(oxv, ox_hbm.at[pl.ds(base, _RS)])
    pltpu.sync_copy(oyv, oy_hbm.at[pl.ds(base, _RS)])
    pltpu.sync_copy(ozv, oz_hbm.at[pl.ds(base, _RS)])

    ps = pstatv[...]
    Rp = [[_extract_lane(ps, 1 + 3 * j + k) for k in range(3)] for j in range(3)]
    tp = [_extract_lane(ps, 10 + j) for j in range(3)]
    Rn = [[R[j][0] * Rp[0][k] + R[j][1] * Rp[1][k] + R[j][2] * Rp[2][k]
           for k in range(3)] for j in range(3)]
    tn = [R[j][0] * tp[0] + R[j][1] * tp[1] + R[j][2] * tp[2] + t[j]
          for j in range(3)]

    flat = [mean_error] + [Rn[j][k] for j in range(3) for k in range(3)] + \
        list(tn)
    out = jnp.zeros((16,), _F32)
    for k in range(13):
        out = jnp.where(lanes == k, jnp.full((16,), flat[k], _F32), out)

    @pl.when(sid == 0)
    def _():
        pvec[...] = out
        pltpu.sync_copy(pvec, stats_hbm)


def _sc_fit_call(sx, sy, sz, dx, dy, dz, bidx, dist, pstat):
    mesh = plsc.VectorSubcoreMesh(core_axis_name="c", subcore_axis_name="s",
                                  num_cores=1, num_subcores=_NSUB)
    f = pl.kernel(
        _sc_fit_body,
        out_type=[
            jax.ShapeDtypeStruct((_N,), _F32),
            jax.ShapeDtypeStruct((_N,), _F32),
            jax.ShapeDtypeStruct((_N,), _F32),
            jax.ShapeDtypeStruct((16,), _F32),
        ],
        mesh=mesh,
        scratch_types=[
            pltpu.VMEM((_N,), _F32),
            pltpu.VMEM((_N,), _F32),
            pltpu.VMEM((_N,), _F32),
            pltpu.VMEM((_RS,), _F32),
            pltpu.VMEM((_RS,), _F32),
            pltpu.VMEM((_RS,), _F32),
            pltpu.VMEM((_RS,), jnp.int32),
            pltpu.VMEM((_RS,), _F32),
            pltpu.VMEM((16,), _F32),
            pltpu.VMEM_SHARED((_NSUB * 16,), _F32),
            pltpu.VMEM((_NSUB * 16,), _F32),
            pltpu.VMEM((_RS,), _F32),
            pltpu.VMEM((_RS,), _F32),
            pltpu.VMEM((_RS,), _F32),
            pltpu.VMEM((16,), _F32),
        ],
        compiler_params=pltpu.CompilerParams(needs_layout_passes=False),
        interpret=_INTERPRET,
    )
    return f(sx, sy, sz, dx, dy, dz, bidx, dist, pstat)


def kernel(A, B):
    max_iterations = 20
    tolerance = 0.001
    dx = B[:, 0]
    dy = B[:, 1]
    dz = B[:, 2]
    qd = dx * dx + dy * dy + dz * dz
    dstP = jnp.concatenate([B, qd[:, None]], axis=1)

    def cond(c):
        _, _, _, _, _, done, i = c
        return jnp.logical_and(i < max_iterations, jnp.logical_not(done))

    def body(c):
        sx, sy, sz, pstat, prev_error, done, i = c
        bidx, dist = _nn_tc(sx, sy, sz, dstP)
        nx, ny, nz, stats = _sc_fit_call(sx, sy, sz, dx, dy, dz, bidx, dist,
                                         pstat)
        mean_error = stats[0]
        converged = jnp.abs(prev_error - mean_error) < tolerance
        return (nx, ny, nz, stats, mean_error, done | converged, i + 1)

    stat0 = jnp.array([0, 1, 0, 0, 0, 1, 0, 0, 0, 1, 0, 0, 0, 0, 0, 0],
                      dtype=_F32)
    init = (A[:, 0], A[:, 1], A[:, 2], stat0, jnp.zeros((), A.dtype),
            jnp.array(False), jnp.array(0, jnp.int32))
    _, _, _, stats, _, _, _ = lax.while_loop(cond, body, init)

    R = stats[1:10].reshape(3, 3)
    t = stats[10:13]
    T = jnp.eye(4, dtype=A.dtype)
    T = T.at[:3, :3].set(R)
    T = T.at[:3, 3].set(t)
    return T
